# baseline (speedup 1.0000x reference)
import functools

import jax
import jax.numpy as jnp
from jax import lax
from jax.experimental import pallas as pl
from jax.experimental.pallas import tpu as pltpu
from jax.experimental.pallas import tpu_sc as plsc

_C = 3
_T = 128
_NUM_SAMPLES = 32
_ROW = 224 * 224
_ROWS_OUT = _C * _NUM_SAMPLES
_NWORKERS = 32
_RPW = _ROWS_OUT // _NWORKERS


def _src_row(r):
    c = r // _NUM_SAMPLES
    i = r - c * _NUM_SAMPLES
    return c * _T + (i * 127) // 31


def _body(x_hbm, out_hbm, buf0, buf1, sem0, sem1):
    wid = lax.axis_index("s") * 2 + lax.axis_index("c")
    bufs = (buf0, buf1)
    sems = (sem0, sem1)

    copies = [None] * _RPW
    writes = [None] * _RPW
    for j in range(min(2, _RPW)):
        r = wid * _RPW + j
        copies[j] = pltpu.async_copy(x_hbm.at[_src_row(r)], bufs[j % 2], sems[j % 2])
    for j in range(_RPW):
        r = wid * _RPW + j
        copies[j].wait()
        writes[j] = pltpu.async_copy(bufs[j % 2], out_hbm.at[r], sems[j % 2])
        nxt = j + 2
        if nxt < _RPW:
            rn = wid * _RPW + nxt
            writes[j].wait()
            copies[nxt] = pltpu.async_copy(
                x_hbm.at[_src_row(rn)], bufs[nxt % 2], sems[nxt % 2]
            )
    for j in range(_RPW):
        if writes[j] is not None and (j + 2) >= _RPW:
            writes[j].wait()


@jax.jit
def _subsample(x2d):
    mesh = plsc.VectorSubcoreMesh(core_axis_name="c", subcore_axis_name="s")
    kern = functools.partial(
        pl.kernel,
        mesh=mesh,
        out_type=jax.ShapeDtypeStruct((_ROWS_OUT, _ROW), jnp.float32),
        scratch_types=[
            pltpu.VMEM((_ROW,), jnp.float32),
            pltpu.VMEM((_ROW,), jnp.float32),
            pltpu.SemaphoreType.DMA,
            pltpu.SemaphoreType.DMA,
        ],
    )(_body)
    return kern(x2d)


def kernel(x):
    x2d = x.reshape(_C * _T, _ROW)
    out = _subsample(x2d)
    return out.reshape(_C, _NUM_SAMPLES, 224, 224)


# speedup vs baseline: 2.0580x; 2.0580x over previous
import functools

import jax
import jax.numpy as jnp
from jax import lax
from jax.experimental import pallas as pl
from jax.experimental.pallas import tpu as pltpu
from jax.experimental.pallas import tpu_sc as plsc

_C = 3
_T = 128
_NUM_SAMPLES = 32
_H = 224
_W = 224
_ROWS_OUT = _C * _NUM_SAMPLES
_NWORKERS = 32
_RPW = _ROWS_OUT // _NWORKERS


def _src_row(r):
    c = r // _NUM_SAMPLES
    i = r - c * _NUM_SAMPLES
    return c * _T + (i * 127) // 31


def _body(x_hbm, out_hbm, buf0, buf1, sem0, sem1):
    wid = lax.axis_index("s") * 2 + lax.axis_index("c")
    bufs = (buf0, buf1)
    sems = (sem0, sem1)

    copies = [None] * _RPW
    writes = [None] * _RPW
    for j in range(min(2, _RPW)):
        r = wid * _RPW + j
        copies[j] = pltpu.async_copy(x_hbm.at[_src_row(r)], bufs[j % 2], sems[j % 2])
    for j in range(_RPW):
        r = wid * _RPW + j
        copies[j].wait()
        writes[j] = pltpu.async_copy(bufs[j % 2], out_hbm.at[r], sems[j % 2])
        nxt = j + 2
        if nxt < _RPW:
            rn = wid * _RPW + nxt
            writes[j].wait()
            copies[nxt] = pltpu.async_copy(
                x_hbm.at[_src_row(rn)], bufs[nxt % 2], sems[nxt % 2]
            )
    for j in range(_RPW):
        if writes[j] is not None and (j + 2) >= _RPW:
            writes[j].wait()


@jax.jit
def _subsample(x3d):
    mesh = plsc.VectorSubcoreMesh(core_axis_name="c", subcore_axis_name="s")
    kern = functools.partial(
        pl.kernel,
        mesh=mesh,
        out_type=jax.ShapeDtypeStruct((_ROWS_OUT, _H, _W), jnp.float32),
        scratch_types=[
            pltpu.VMEM((_H, _W), jnp.float32),
            pltpu.VMEM((_H, _W), jnp.float32),
            pltpu.SemaphoreType.DMA,
            pltpu.SemaphoreType.DMA,
        ],
        compiler_params=pltpu.CompilerParams(use_tc_tiling_on_sc=True),
    )(_body)
    return kern(x3d)


def kernel(x):
    x3d = x.reshape(_C * _T, _H, _W)
    out = _subsample(x3d)
    return out.reshape(_C, _NUM_SAMPLES, _H, _W)
